# baseline (speedup 1.0000x reference)
import functools

import jax
import jax.numpy as jnp
from jax import lax
from jax.experimental import pallas as pl
from jax.experimental.pallas import tpu as pltpu
from jax.experimental.pallas import tpu_sc as plsc

_VOCAB = 1000000
_MAXLEN = 200
_DIM = 64
_BATCH = 4096

_NC, _NS = 2, 16
_NW = _NC * _NS
_B = _BATCH * _MAXLEN
_PER_W = _B // _NW
_ROWS_PER_CHUNK = 2
_CHUNK = _ROWS_PER_CHUNK * _MAXLEN
_NCHUNK = _PER_W // _CHUNK
_GATHER = 80
_NGATHER = _CHUNK // _GATHER


def _body(x_hbm, tok_hbm, pos_hbm, out_hbm, idx_v, rows_v, pos_v, gsem, wsem):
    wid = lax.axis_index("s") * _NC + lax.axis_index("c")
    base = wid * _PER_W

    pltpu.sync_copy(x_hbm.at[pl.ds(base, _PER_W)], idx_v)
    pltpu.sync_copy(pos_hbm, pos_v)

    def chunk_body(c, carry):
        tok0 = c * _CHUNK
        copies = [
            pltpu.async_copy(
                tok_hbm.at[idx_v.at[pl.ds(tok0 + g * _GATHER, _GATHER)]],
                rows_v.at[pl.ds(g * _GATHER, _GATHER)],
                gsem,
            )
            for g in range(_NGATHER)
        ]
        for cp in copies:
            cp.wait()

        def add_body(s, carry2):
            for r in range(_ROWS_PER_CHUNK):
                t = r * _MAXLEN + s
                for k in range(_DIM // 16):
                    d = pl.ds(k * 16, 16)
                    rows_v[t, d] = rows_v[t, d] + pos_v[s, d]
            return carry2

        lax.fori_loop(0, _MAXLEN, add_body, 0)

        pltpu.async_copy(rows_v, out_hbm.at[pl.ds(base + tok0, _CHUNK)], wsem).wait()
        return carry

    lax.fori_loop(0, _NCHUNK, chunk_body, 0)


@jax.jit
def _embed(x_flat, token_table, pos_table):
    mesh = plsc.VectorSubcoreMesh(core_axis_name="c", subcore_axis_name="s")
    run = pl.kernel(
        _body,
        out_type=jax.ShapeDtypeStruct((_B, _DIM), jnp.float32),
        mesh=mesh,
        scratch_types=[
            pltpu.VMEM((_PER_W,), jnp.int32),
            pltpu.VMEM((_CHUNK, _DIM), jnp.float32),
            pltpu.VMEM((_MAXLEN, _DIM), jnp.float32),
            pltpu.SemaphoreType.DMA,
            pltpu.SemaphoreType.DMA,
        ],
        compiler_params=pltpu.CompilerParams(use_tc_tiling_on_sc=False),
    )
    return run(x_flat, token_table, pos_table)


def kernel(x, token_table, pos_table):
    x_flat = x.reshape(-1).astype(jnp.int32)
    out = _embed(x_flat, token_table, pos_table)
    return out.reshape(_BATCH, _MAXLEN, _DIM)


# speedup vs baseline: 1.1104x; 1.1104x over previous
import functools

import jax
import jax.numpy as jnp
from jax import lax
from jax.experimental import pallas as pl
from jax.experimental.pallas import tpu as pltpu
from jax.experimental.pallas import tpu_sc as plsc

_VOCAB = 1000000
_MAXLEN = 200
_DIM = 64
_BATCH = 4096

_NC, _NS = 2, 16
_NW = _NC * _NS
_B = _BATCH * _MAXLEN
_PER_W = _B // _NW
_CHUNK = _MAXLEN
_NCHUNK = _PER_W // _CHUNK
_NBUF = 4
_GSIZES = (128, 72)


def _body(x_hbm, tok_hbm, pos_hbm, out_hbm,
          idx_v, pos_v, bufs, gsems, wsems):
    wid = lax.axis_index("s") * _NC + lax.axis_index("c")
    base = wid * _PER_W

    pltpu.sync_copy(x_hbm.at[pl.ds(base, _PER_W)], idx_v)
    pltpu.sync_copy(pos_hbm, pos_v)

    def issue_gathers(c, b):
        tok0 = c * _CHUNK
        off = 0
        for gs in _GSIZES:
            pltpu.async_copy(
                tok_hbm.at[idx_v.at[pl.ds(tok0 + off, gs)]],
                bufs[b].at[pl.ds(off, gs)],
                gsems[b],
            )
            off += gs

    def wait_gathers(c, b):
        tok0 = c * _CHUNK
        off = 0
        for gs in _GSIZES:
            pltpu.make_async_copy(
                tok_hbm.at[idx_v.at[pl.ds(tok0 + off, gs)]],
                bufs[b].at[pl.ds(off, gs)],
                gsems[b],
            ).wait()
            off += gs

    def issue_write(c, b):
        pltpu.async_copy(bufs[b], out_hbm.at[pl.ds(base + c * _CHUNK, _CHUNK)],
                         wsems[b])

    def wait_write(c, b):
        pltpu.make_async_copy(bufs[b],
                              out_hbm.at[pl.ds(base + c * _CHUNK, _CHUNK)],
                              wsems[b]).wait()

    issue_gathers(0, 0)
    issue_gathers(1, 1)

    def outer(i, carry):
        for j in range(_NBUF):
            c = i * _NBUF + j
            b = j
            b2 = (j + 2) % _NBUF
            pl.when(c >= 2)(lambda: wait_write(c - 2, b2))
            pl.when(c + 2 < _NCHUNK)(lambda: issue_gathers(c + 2, b2))

            wait_gathers(c, b)

            def add_body(u, carry2):
                for v in range(4):
                    t = u * 4 + v
                    for k in range(_DIM // 16):
                        d = pl.ds(k * 16, 16)
                        bufs[b][t, d] = bufs[b][t, d] + pos_v[t, d]
                return carry2

            lax.fori_loop(0, _CHUNK // 4, add_body, 0)

            issue_write(c, b)
        return carry

    lax.fori_loop(0, _NCHUNK // _NBUF, outer, 0)

    wait_write(_NCHUNK - 2, (_NCHUNK - 2) % _NBUF)
    wait_write(_NCHUNK - 1, (_NCHUNK - 1) % _NBUF)


@jax.jit
def _embed(x_flat, token_table, pos_table):
    mesh = plsc.VectorSubcoreMesh(core_axis_name="c", subcore_axis_name="s")

    def wrapped(x_hbm, tok_hbm, pos_hbm, out_hbm, idx_v, pos_v,
                b0, b1, b2, b3, g0, g1, g2, g3, w0, w1, w2, w3):
        _body(x_hbm, tok_hbm, pos_hbm, out_hbm, idx_v, pos_v,
              (b0, b1, b2, b3), (g0, g1, g2, g3), (w0, w1, w2, w3))

    run = pl.kernel(
        wrapped,
        out_type=jax.ShapeDtypeStruct((_B, _DIM), jnp.float32),
        mesh=mesh,
        scratch_types=(
            [pltpu.VMEM((_PER_W,), jnp.int32),
             pltpu.VMEM((_MAXLEN, _DIM), jnp.float32)]
            + [pltpu.VMEM((_CHUNK, _DIM), jnp.float32)] * _NBUF
            + [pltpu.SemaphoreType.DMA] * (2 * _NBUF)
        ),
        compiler_params=pltpu.CompilerParams(use_tc_tiling_on_sc=False),
    )
    return run(x_flat, token_table, pos_table)


def kernel(x, token_table, pos_table):
    x_flat = x.reshape(-1).astype(jnp.int32)
    out = _embed(x_flat, token_table, pos_table)
    return out.reshape(_BATCH, _MAXLEN, _DIM)
